# baseline (speedup 1.0000x reference)
import jax
import jax.numpy as jnp
import numpy as np
from jax.experimental import pallas as pl
from jax.experimental.pallas import tpu as pltpu

_NUM_EMB = 1024
_EMB_DIM = 64
_SEQ = 1024


_SLABS = 4


def _vq_slab(xt, wt, icol, w_norm):
    i_norm = jnp.sum(xt * xt, axis=0, keepdims=True)

    neg_mm = jax.lax.dot_general(
        wt * -2.0, xt, (((0,), (0,)), ((), ())),
        preferred_element_type=jnp.float32)

    _C = 256
    mn = None
    idx = None
    for c in range(_NUM_EMB // _C):
        wn_c = jax.lax.slice(w_norm, (c * _C, 0), ((c + 1) * _C, 1))
        nm_c = jax.lax.slice(neg_mm, (c * _C, 0), ((c + 1) * _C, _SEQ))
        d_c = (wn_c + i_norm) + nm_c
        cmn = jnp.min(d_c, axis=0, keepdims=True)
        icol_c = jax.lax.slice(icol, (c * _C, 0), ((c + 1) * _C, 1))
        cand_c = jnp.where(d_c == cmn, icol_c, jnp.float32(_NUM_EMB))
        cidx = jnp.min(cand_c, axis=0, keepdims=True)
        if mn is None:
            mn, idx = cmn, cidx
        else:
            upd = cmn < mn
            idx = jnp.where(upd, cidx, idx)
            mn = jnp.minimum(mn, cmn)

    one_hot = (icol == idx).astype(jnp.float32)
    q = jax.lax.dot_general(
        wt, one_hot, (((1,), (0,)), ((), ())),
        preferred_element_type=jnp.float32)

    return xt + (q - xt), jnp.sum((xt - q) ** 2)


def _vq_block_kernel(xt_ref, wt_ref, icol_ref, out_ref, loss_ref):
    wt = wt_ref[...]
    icol = icol_ref[...]

    ones_col = jnp.ones((_EMB_DIM, 1), dtype=jnp.float32)
    w_norm = jax.lax.dot_general(
        wt * wt, ones_col, (((0,), (0,)), ((), ())),
        preferred_element_type=jnp.float32)

    loss = jnp.zeros((), dtype=jnp.float32)
    for s in range(_SLABS):
        xt = xt_ref[s]
        out, part = _vq_slab(xt, wt, icol, w_norm)
        out_ref[s] = out
        loss = loss + part

    i = pl.program_id(0)
    prev = jnp.where(i == 0, jnp.zeros((1, 1, 1), jnp.float32),
                     loss_ref[...])
    acc = prev + loss.reshape(1, 1, 1)
    last = i == pl.num_programs(0) - 1
    loss_ref[...] = jnp.where(last, acc * jnp.float32(0.25 / (2.0 ** 20)),
                              acc)


@jax.jit
def kernel(input, W):
    shape = input.shape
    grid = shape[0] // _SLABS

    xt = jnp.swapaxes(input, 1, 2)
    wt = W.T
    icol = jnp.asarray(np.arange(_NUM_EMB, dtype=np.float32)[:, None])

    out_t, loss_parts = pl.pallas_call(
        _vq_block_kernel,
        grid=(grid,),
        in_specs=[
            pl.BlockSpec((_SLABS, _EMB_DIM, _SEQ), lambda i: (i, 0, 0)),
            pl.BlockSpec((_EMB_DIM, _NUM_EMB), lambda i: (0, 0)),
            pl.BlockSpec((_NUM_EMB, 1), lambda i: (0, 0)),
        ],
        out_specs=[
            pl.BlockSpec((_SLABS, _EMB_DIM, _SEQ), lambda i: (i, 0, 0)),
            pl.BlockSpec((1, 1, 1), lambda i: (0, 0, 0)),
        ],
        out_shape=[
            jax.ShapeDtypeStruct((shape[0], _EMB_DIM, _SEQ), jnp.float32),
            jax.ShapeDtypeStruct((1, 1, 1), jnp.float32),
        ],
        compiler_params=pltpu.CompilerParams(
            dimension_semantics=("arbitrary",)),
    )(xt, wt, icol)

    return (jnp.swapaxes(out_t, 1, 2), loss_parts.reshape(()))
